# Initial kernel scaffold; baseline (speedup 1.0000x reference)
import jax
import jax.numpy as jnp
from jax.experimental import pallas as pl


def kernel(elements_one_hot, psi, batch_segments, num_graphs, Wq, Wk, Wv, W1, W2):
    raise NotImplementedError("write your pallas kernel here")


# speedup vs baseline: 5.9827x; 5.9827x over previous
import functools

import jax
import jax.numpy as jnp
import numpy as np
from jax.experimental import pallas as pl
from jax.experimental.pallas import tpu as pltpu
from jax.experimental.pallas import tpu_sc as plsc

N_ATOMS = 100000
NUM_FEATURES = 128
NUM_ELEMENTS = 118
NUM_GRAPHS = 1024

N_TILES = 16
CHUNK = 6272
N_PAD = N_TILES * CHUNK
VECS = CHUNK // 16
G_VECS = NUM_GRAPHS // 16

B1 = 2000
B2 = 2000

_INV_SQRT_D = float(1.0 / np.sqrt(NUM_FEATURES))
_EPS = 1e-6


def _y_body(eoh_ref, wq_ref, wk_ref, y_ref):
    w = jnp.dot(wk_ref[0:1, :], wq_ref[...], preferred_element_type=jnp.float32)
    s = jnp.sum(eoh_ref[...] * w, axis=1, keepdims=True) * _INV_SQRT_D
    y_ref[...] = jax.nn.softplus(s)


def _mlp_body(att_ref, wv_ref, w1t_ref, w2t_ref, out_ref):
    a = att_ref[...]
    v_att = a * wv_ref[0:1, :]
    h = jax.nn.silu(v_att)
    h = jnp.dot(h, w1t_ref[...], preferred_element_type=jnp.float32)
    h = jax.nn.silu(h)
    h = jnp.dot(h, w2t_ref[...], preferred_element_type=jnp.float32)
    out_ref[...] = v_att + h


_sc_mesh = plsc.VectorSubcoreMesh(
    core_axis_name="c", subcore_axis_name="s", num_cores=1
)


@functools.partial(
    pl.kernel,
    out_type=jax.ShapeDtypeStruct((N_PAD,), jnp.float32),
    mesh=_sc_mesh,
    compiler_params=pltpu.CompilerParams(needs_layout_passes=False),
    scratch_types=[
        pltpu.VMEM((CHUNK,), jnp.float32),
        pltpu.VMEM((CHUNK,), jnp.int32),
        pltpu.VMEM((NUM_GRAPHS,), jnp.float32),
        pltpu.VMEM((NUM_GRAPHS,), jnp.float32),
        pltpu.VMEM((N_TILES, NUM_GRAPHS), jnp.float32),
        pltpu.VMEM_SHARED((N_TILES, NUM_GRAPHS), jnp.float32),
    ],
)
def _att_sc(y_hbm, seg_hbm, psi_hbm, att_hbm, y_v, seg_v, acc_v, psi_v,
            all_v, shared):
    sid = jax.lax.axis_index("s")
    base = sid * CHUNK
    pltpu.sync_copy(y_hbm.at[pl.ds(base, CHUNK)], y_v)
    pltpu.sync_copy(seg_hbm.at[pl.ds(base, CHUNK)], seg_v)
    pltpu.sync_copy(psi_hbm, psi_v)

    zero16 = jnp.zeros((16,), jnp.float32)

    def zero_step(j, c):
        acc_v[pl.ds(j * 16, 16)] = zero16
        return c

    jax.lax.fori_loop(0, G_VECS, zero_step, 0)

    def acc_step(i, c):
        idx = seg_v[pl.ds(i * 16, 16)]
        val = y_v[pl.ds(i * 16, 16)]
        plsc.addupdate_scatter(acc_v, [idx], val)
        return c

    jax.lax.fori_loop(0, VECS, acc_step, 0)

    pltpu.sync_copy(acc_v, shared.at[sid])
    plsc.subcore_barrier()
    pltpu.sync_copy(shared, all_v)

    def ratio_step(j, c):
        s = all_v[0, pl.ds(j * 16, 16)]
        for r in range(1, N_TILES):
            s = s + all_v[r, pl.ds(j * 16, 16)]
        acc_v[pl.ds(j * 16, 16)] = psi_v[pl.ds(j * 16, 16)] / (s + _EPS)
        return c

    jax.lax.fori_loop(0, G_VECS, ratio_step, 0)

    def att_step(i, c):
        idx = seg_v[pl.ds(i * 16, 16)]
        rat = plsc.load_gather(acc_v, [idx])
        y_v[pl.ds(i * 16, 16)] = rat * y_v[pl.ds(i * 16, 16)]
        return c

    jax.lax.fori_loop(0, VECS, att_step, 0)
    pltpu.sync_copy(y_v, att_hbm.at[pl.ds(base, CHUNK)])


def kernel(elements_one_hot, psi, batch_segments, num_graphs, Wq, Wk, Wv, W1, W2):
    n = elements_one_hot.shape[0]

    y = pl.pallas_call(
        _y_body,
        grid=(n // B1,),
        in_specs=[
            pl.BlockSpec((B1, NUM_ELEMENTS), lambda i: (i, 0)),
            pl.BlockSpec((NUM_FEATURES, NUM_ELEMENTS), lambda i: (0, 0)),
            pl.BlockSpec((2, NUM_FEATURES), lambda i: (0, 0)),
        ],
        out_specs=pl.BlockSpec((B1, 1), lambda i: (i, 0)),
        out_shape=jax.ShapeDtypeStruct((n, 1), jnp.float32),
    )(elements_one_hot, Wq, Wk)

    pad = N_PAD - n
    y_pad = jnp.concatenate([y[:, 0], jnp.zeros((pad,), jnp.float32)])
    seg_pad = jnp.concatenate(
        [batch_segments.astype(jnp.int32), jnp.zeros((pad,), jnp.int32)]
    )

    att = _att_sc(y_pad, seg_pad, psi)
    att2 = att[:n].reshape(n, 1)

    out = pl.pallas_call(
        _mlp_body,
        grid=(n // B2,),
        in_specs=[
            pl.BlockSpec((B2, 1), lambda i: (i, 0)),
            pl.BlockSpec((2, NUM_FEATURES), lambda i: (0, 0)),
            pl.BlockSpec((NUM_FEATURES, NUM_FEATURES), lambda i: (0, 0)),
            pl.BlockSpec((NUM_FEATURES, NUM_FEATURES), lambda i: (0, 0)),
        ],
        out_specs=pl.BlockSpec((B2, NUM_FEATURES), lambda i: (i, 0)),
        out_shape=jax.ShapeDtypeStruct((n, NUM_FEATURES), jnp.float32),
    )(att2, Wv, W1.T, W2.T)

    return out
